# baseline (speedup 1.0000x reference)
import functools

import jax
import jax.numpy as jnp
from jax import lax
from jax.experimental import pallas as pl
from jax.experimental.pallas import tpu as pltpu
from jax.experimental.pallas import tpu_sc as plsc

_NC = 2
_NS = 16
_NW = _NC * _NS
_K = 32
_NBUF = 3
_SC_BATCHES = 1


def _tc_add(x_ref, e_ref, o_ref):
    o_ref[...] = x_ref[...] + e_ref[...]


def _sc_body(row0, seq_len, dim, in_hbm, emb_hbm, out_hbm,
             idx_v, bufs, asems, wsems):
    wid = lax.axis_index("s") * _NC + lax.axis_index("c")
    sc_rows = _SC_BATCHES * seq_len
    rows_per_w = sc_rows // _NW
    base = wid * rows_per_w
    ebase = lax.rem(base, seq_len)
    nchunks = rows_per_w // _K

    adescs = [None] * nchunks
    wdescs = [None] * nchunks
    for c in range(nchunks):
        s = c % _NBUF
        if c >= _NBUF:
            wdescs[c - _NBUF].wait()
        for j in range(_K // 16):
            idx_v[s][pl.ds(j * 16, 16)] = (row0 + base + c * _K + j * 16) + lax.iota(
                jnp.int32, 16)
        pltpu.sync_copy(emb_hbm.at[pl.ds(ebase + c * _K, _K)], bufs[s])
        adescs[c] = pltpu.async_copy(in_hbm.at[idx_v[s]], bufs[s], asems[s],
                                     add=True)
        if c >= 1:
            p = c - 1
            adescs[p].wait()
            wdescs[p] = pltpu.async_copy(
                bufs[p % _NBUF], out_hbm.at[pl.ds(base + p * _K, _K)],
                wsems[p % _NBUF])
    last = nchunks - 1
    adescs[last].wait()
    wdescs[last] = pltpu.async_copy(
        bufs[last % _NBUF], out_hbm.at[pl.ds(base + last * _K, _K)],
        wsems[last % _NBUF])
    for c in range(max(0, nchunks - _NBUF), nchunks):
        wdescs[c].wait()


def kernel(inputs, embeddings):
    batch, seq_len, dim = inputs.shape
    rows = batch * seq_len
    in_flat = inputs.reshape(rows, dim)
    pos = embeddings[:seq_len]
    tc_batches = batch - _SC_BATCHES
    row0 = tc_batches * seq_len

    tc_out = pl.pallas_call(
        _tc_add,
        grid=(tc_batches,),
        in_specs=[
            pl.BlockSpec((seq_len, dim), lambda j: (j, 0)),
            pl.BlockSpec((seq_len, dim), lambda j: (0, 0)),
        ],
        out_specs=pl.BlockSpec((seq_len, dim), lambda j: (j, 0)),
        out_shape=jax.ShapeDtypeStruct((row0, dim), inputs.dtype),
        compiler_params=pltpu.CompilerParams(
            dimension_semantics=("arbitrary",),
        ),
    )(in_flat, pos)

    mesh = plsc.VectorSubcoreMesh(core_axis_name="c", subcore_axis_name="s")
    sc_k = pl.kernel(
        functools.partial(_sc_body, row0, seq_len, dim),
        out_type=jax.ShapeDtypeStruct((_SC_BATCHES * seq_len, dim), inputs.dtype),
        mesh=mesh,
        scratch_types=[
            [pltpu.VMEM((_K,), jnp.int32) for _ in range(_NBUF)],
            [pltpu.VMEM((_K, dim), jnp.float32) for _ in range(_NBUF)],
            [pltpu.SemaphoreType.DMA for _ in range(_NBUF)],
            [pltpu.SemaphoreType.DMA for _ in range(_NBUF)],
        ],
    )
    sc_out = sc_k(in_flat, pos)
    return jnp.concatenate([tc_out, sc_out], axis=0).reshape(batch, seq_len, dim)


# speedup vs baseline: 2.7516x; 2.7516x over previous
import functools

import jax
import jax.numpy as jnp
from jax.experimental import pallas as pl
from jax.experimental.pallas import tpu as pltpu

_C = 256
_NB = 8


def _body(rows, seq_len, dim, in_hbm, emb_hbm, out_hbm,
          emb_v, in_bufs, out_bufs, esem, isems, osems):
    nch = rows // _C
    edesc = pltpu.make_async_copy(emb_hbm, emb_v, esem)
    edesc.start()
    in_descs = [None] * nch
    out_descs = [None] * nch
    for c in range(min(_NB, nch)):
        in_descs[c] = pltpu.make_async_copy(
            in_hbm.at[pl.ds(c * _C, _C)], in_bufs[c % _NB], isems[c % _NB])
        in_descs[c].start()
    edesc.wait()
    for c in range(nch):
        s = c % _NB
        in_descs[c].wait()
        if c >= _NB:
            out_descs[c - _NB].wait()
        out_bufs[s][...] = in_bufs[s][...] + emb_v[pl.ds((c * _C) % seq_len, _C), :]
        out_descs[c] = pltpu.make_async_copy(
            out_bufs[s], out_hbm.at[pl.ds(c * _C, _C)], osems[s])
        out_descs[c].start()
        if c + _NB < nch:
            in_descs[c + _NB] = pltpu.make_async_copy(
                in_hbm.at[pl.ds((c + _NB) * _C, _C)], in_bufs[s], isems[s])
            in_descs[c + _NB].start()
    for c in range(max(0, nch - _NB), nch):
        out_descs[c].wait()


def kernel(inputs, embeddings):
    batch, seq_len, dim = inputs.shape
    rows = batch * seq_len
    in_flat = inputs.reshape(rows, dim)
    pos = embeddings[:seq_len]
    out = pl.pallas_call(
        functools.partial(_body, rows, seq_len, dim),
        in_specs=[
            pl.BlockSpec(memory_space=pl.ANY),
            pl.BlockSpec(memory_space=pl.ANY),
        ],
        out_specs=pl.BlockSpec(memory_space=pl.ANY),
        out_shape=jax.ShapeDtypeStruct((rows, dim), inputs.dtype),
        scratch_shapes=[
            pltpu.VMEM((seq_len, dim), jnp.float32),
            [pltpu.VMEM((_C, dim), jnp.float32) for _ in range(_NB)],
            [pltpu.VMEM((_C, dim), jnp.float32) for _ in range(_NB)],
            pltpu.SemaphoreType.DMA,
            [pltpu.SemaphoreType.DMA for _ in range(_NB)],
            [pltpu.SemaphoreType.DMA for _ in range(_NB)],
        ],
        compiler_params=pltpu.CompilerParams(
            vmem_limit_bytes=100 * 1024 * 1024,
        ),
    )(in_flat, pos)
    return out.reshape(batch, seq_len, dim)


# speedup vs baseline: 2.7675x; 1.0058x over previous
import functools

import jax
import jax.numpy as jnp
from jax.experimental import pallas as pl
from jax.experimental.pallas import tpu as pltpu

_C = 1024
_NB = 3


def _body(rows, seq_len, dim, in_hbm, emb_hbm, out_hbm,
          emb_v, in_bufs, out_bufs, esem, isems, osems):
    nch = rows // _C
    edesc = pltpu.make_async_copy(emb_hbm, emb_v, esem)
    edesc.start()
    in_descs = [None] * nch
    out_descs = [None] * nch
    for c in range(min(_NB, nch)):
        in_descs[c] = pltpu.make_async_copy(
            in_hbm.at[pl.ds(c * _C, _C)], in_bufs[c % _NB], isems[c % _NB])
        in_descs[c].start()
    edesc.wait()
    for c in range(nch):
        s = c % _NB
        in_descs[c].wait()
        if c >= _NB:
            out_descs[c - _NB].wait()
        out_bufs[s][...] = in_bufs[s][...] + emb_v[pl.ds((c * _C) % seq_len, _C), :]
        out_descs[c] = pltpu.make_async_copy(
            out_bufs[s], out_hbm.at[pl.ds(c * _C, _C)], osems[s])
        out_descs[c].start()
        if c + _NB < nch:
            in_descs[c + _NB] = pltpu.make_async_copy(
                in_hbm.at[pl.ds((c + _NB) * _C, _C)], in_bufs[s], isems[s])
            in_descs[c + _NB].start()
    for c in range(max(0, nch - _NB), nch):
        out_descs[c].wait()


def kernel(inputs, embeddings):
    batch, seq_len, dim = inputs.shape
    rows = batch * seq_len
    in_flat = inputs.reshape(rows, dim)
    pos = embeddings[:seq_len]
    out = pl.pallas_call(
        functools.partial(_body, rows, seq_len, dim),
        in_specs=[
            pl.BlockSpec(memory_space=pl.ANY),
            pl.BlockSpec(memory_space=pl.ANY),
        ],
        out_specs=pl.BlockSpec(memory_space=pl.ANY),
        out_shape=jax.ShapeDtypeStruct((rows, dim), inputs.dtype),
        scratch_shapes=[
            pltpu.VMEM((seq_len, dim), jnp.float32),
            [pltpu.VMEM((_C, dim), jnp.float32) for _ in range(_NB)],
            [pltpu.VMEM((_C, dim), jnp.float32) for _ in range(_NB)],
            pltpu.SemaphoreType.DMA,
            [pltpu.SemaphoreType.DMA for _ in range(_NB)],
            [pltpu.SemaphoreType.DMA for _ in range(_NB)],
        ],
        compiler_params=pltpu.CompilerParams(
            vmem_limit_bytes=100 * 1024 * 1024,
        ),
    )(in_flat, pos)
    return out.reshape(batch, seq_len, dim)


# speedup vs baseline: 2.8080x; 1.0147x over previous
import functools

import jax
import jax.numpy as jnp
from jax.experimental import pallas as pl
from jax.experimental.pallas import tpu as pltpu

_C = 2048
_NB = 3


def _body(rows, seq_len, dim, in_hbm, emb_hbm, out_hbm,
          emb_v, in_bufs, out_bufs, esem, isems, osems):
    nch = rows // _C
    edesc = pltpu.make_async_copy(emb_hbm, emb_v, esem)
    edesc.start()
    in_descs = [None] * nch
    out_descs = [None] * nch
    for c in range(min(_NB, nch)):
        in_descs[c] = pltpu.make_async_copy(
            in_hbm.at[pl.ds(c * _C, _C)], in_bufs[c % _NB], isems[c % _NB])
        in_descs[c].start()
    edesc.wait()
    for c in range(nch):
        s = c % _NB
        in_descs[c].wait()
        if c >= _NB:
            out_descs[c - _NB].wait()
        out_bufs[s][...] = in_bufs[s][...] + emb_v[pl.ds((c * _C) % seq_len, _C), :]
        out_descs[c] = pltpu.make_async_copy(
            out_bufs[s], out_hbm.at[pl.ds(c * _C, _C)], osems[s])
        out_descs[c].start()
        if c + _NB < nch:
            in_descs[c + _NB] = pltpu.make_async_copy(
                in_hbm.at[pl.ds((c + _NB) * _C, _C)], in_bufs[s], isems[s])
            in_descs[c + _NB].start()
    for c in range(max(0, nch - _NB), nch):
        out_descs[c].wait()


def kernel(inputs, embeddings):
    batch, seq_len, dim = inputs.shape
    rows = batch * seq_len
    in_flat = inputs.reshape(rows, dim)
    pos = embeddings[:seq_len]
    out = pl.pallas_call(
        functools.partial(_body, rows, seq_len, dim),
        in_specs=[
            pl.BlockSpec(memory_space=pl.ANY),
            pl.BlockSpec(memory_space=pl.ANY),
        ],
        out_specs=pl.BlockSpec(memory_space=pl.ANY),
        out_shape=jax.ShapeDtypeStruct((rows, dim), inputs.dtype),
        scratch_shapes=[
            pltpu.VMEM((seq_len, dim), jnp.float32),
            [pltpu.VMEM((_C, dim), jnp.float32) for _ in range(_NB)],
            [pltpu.VMEM((_C, dim), jnp.float32) for _ in range(_NB)],
            pltpu.SemaphoreType.DMA,
            [pltpu.SemaphoreType.DMA for _ in range(_NB)],
            [pltpu.SemaphoreType.DMA for _ in range(_NB)],
        ],
        compiler_params=pltpu.CompilerParams(
            vmem_limit_bytes=100 * 1024 * 1024,
        ),
    )(in_flat, pos)
    return out.reshape(batch, seq_len, dim)
